# Initial kernel scaffold; baseline (speedup 1.0000x reference)
import jax
import jax.numpy as jnp
from jax.experimental import pallas as pl


def kernel(input, params):
    raise NotImplementedError("write your pallas kernel here")


# speedup vs baseline: 10.3652x; 10.3652x over previous
import functools

import jax
import jax.numpy as jnp
from jax import lax
from jax.experimental import pallas as pl
from jax.experimental.pallas import tpu as pltpu
from jax.experimental.pallas import tpu_sc as plsc

B = 16384
T = 5
N_OBS = 45
E = 32
NH = 4
DH = 8
NL = 2
LAT = 16
K = 512
DEC_D = 128

R = 512


def _lnT(x, s, b):
    m = jnp.mean(x, axis=0, keepdims=True)
    c = x - m
    v = jnp.mean(c * c, axis=0, keepdims=True)
    return c / jnp.sqrt(v + 1e-5) * s + b


def _mmT(w, x):
    return lax.dot_general(w, x, (((0,), (0,)), ((), ())),
                           preferred_element_type=jnp.float32)


def _bf(x):
    return x.astype(jnp.bfloat16).astype(jnp.float32)


def _headsum(a):
    return jnp.sum(a.reshape(NH, DH, a.shape[-1]), axis=1)


def _headbcast(a, r):
    return jnp.broadcast_to(a[:, None, :], (NH, DH, r)).reshape(NH * DH, r)


def _enc_body(xT, wobs, bobs, posT, ln1s, ln1b, qkvw, qkvb, projw, projb,
              ln2s, ln2b, fcw, fcb, fc2w, fc2b, lnfs, lnfb, headw, headb,
              cb, zT_out, idx_out):
    r = R
    h = [None] * T
    for t in range(T):
        h[t] = _mmT(wobs[...], xT[t]) + bobs[...] + posT[:, t:t + 1]

    rt8 = jnp.sqrt(jnp.float32(DH))
    for l in range(NL):
        qw = qkvw[l]
        qb = qkvb[l]
        hn = [_lnT(h[t], ln1s[l], ln1b[l]) for t in range(T)]
        qkv = [_mmT(qw, hn[t]) + qb for t in range(T)]
        q = [_bf(qkv[t][0:E]) for t in range(T)]
        k = [_bf(qkv[t][E:2 * E]) for t in range(T)]
        v = [_bf(qkv[t][2 * E:3 * E]) for t in range(T)]
        y = [None] * T
        for t1 in range(T):
            row = [_headsum(q[t1] * k[t2]) / rt8 for t2 in range(t1 + 1)]
            mx = row[0]
            for s_ in row[1:]:
                mx = jnp.maximum(mx, s_)
            es = [jnp.exp(s_ - mx) for s_ in row]
            den = es[0]
            for e_ in es[1:]:
                den = den + e_
            acc = None
            for t2 in range(t1 + 1):
                w32 = _headbcast(_bf(es[t2] / den), r)
                acc = w32 * v[t2] if acc is None else acc + w32 * v[t2]
            y[t1] = acc
        pw = projw[l]
        pb = projb[l]
        for t in range(T):
            h[t] = h[t] + _mmT(pw, y[t]) + pb
        fw = fcw[l]
        fb = fcb[l]
        f2w = fc2w[l]
        f2b = fc2b[l]
        for t in range(T):
            hn2 = _lnT(h[t], ln2s[l], ln2b[l])
            g = jax.nn.gelu(_mmT(fw, hn2) + fb)
            h[t] = h[t] + _mmT(f2w, g) + f2b

    hn4 = _lnT(h[T - 1], lnfs[...], lnfb[...])
    z = _mmT(headw[...], hn4) + headb[...]
    zT_out[...] = z

    c = cb[...]
    cb2 = jnp.sum(c * c, axis=1, keepdims=True)
    d = cb2 - 2.0 * lax.dot_general(c, z, (((1,), (0,)), ((), ())),
                                    preferred_element_type=jnp.float32)
    dmin = jnp.min(d, axis=0, keepdims=True)
    ii = lax.broadcasted_iota(jnp.int32, d.shape, 0)
    idx_out[...] = jnp.min(jnp.where(d == dmin, ii, K), axis=0, keepdims=True)


def _ln_row(x, s, b):
    m = jnp.mean(x, axis=1, keepdims=True)
    c = x - m
    v = jnp.mean(c * c, axis=1, keepdims=True)
    return c / jnp.sqrt(v + 1e-5) * s + b


def _elu(x):
    return jnp.where(x > 0, x, jnp.exp(x) - 1.0)


def _table_body(cb, d0w, d0b, d0s, d0bb, d1w, d1b, d1s, d1bb, d2w, d2b, out):
    c = cb[...]
    h = jnp.dot(c, d0w[...], preferred_element_type=jnp.float32) + d0b[...]
    h = _elu(_ln_row(h, d0s[...], d0bb[...]))
    h = jnp.dot(h, d1w[...], preferred_element_type=jnp.float32) + d1b[...]
    h = _elu(_ln_row(h, d1s[...], d1bb[...]))
    o = jnp.dot(h, d2w[...], preferred_element_type=jnp.float32) + d2b[...]
    out[...] = jnp.concatenate(
        [c, o, jnp.zeros((K, DEC_D - LAT - N_OBS), jnp.float32)], axis=1)


def _full(shape):
    rank = len(shape)
    return pl.BlockSpec(shape, lambda *_, _r=rank: (0,) * _r)


def _encode(xT, p):
    grid = (B // R,)
    col2 = lambda a: a.reshape(a.shape[0], a.shape[1], 1)
    in_specs = [pl.BlockSpec((T, N_OBS, R), lambda i: (0, 0, i))]
    args = [xT]

    def add(a, spec_shape=None):
        args.append(a)
        in_specs.append(_full(a.shape))

    add(p['W_obs'])
    add(p['b_obs'].reshape(E, 1))
    add(p['pos'][:T].T)
    add(col2(p['ln1_s']))
    add(col2(p['ln1_b']))
    add(p['qkv_w'])
    add(col2(p['qkv_b']))
    add(p['proj_w'])
    add(col2(p['proj_b']))
    add(col2(p['ln2_s']))
    add(col2(p['ln2_b']))
    add(p['fc_w'])
    add(col2(p['fc_b']))
    add(p['fc2_w'])
    add(col2(p['fc2_b']))
    add(p['lnf_s'].reshape(E, 1))
    add(p['lnf_b'].reshape(E, 1))
    add(p['head_w'])
    add(p['head_b'].reshape(LAT, 1))
    add(p['codebook'])

    zT, idx = pl.pallas_call(
        _enc_body,
        grid=grid,
        in_specs=in_specs,
        out_specs=[pl.BlockSpec((LAT, R), lambda i: (0, i)),
                   pl.BlockSpec((1, R), lambda i: (0, i))],
        out_shape=[jax.ShapeDtypeStruct((LAT, B), jnp.float32),
                   jax.ShapeDtypeStruct((1, B), jnp.int32)],
        compiler_params=pltpu.CompilerParams(
            dimension_semantics=("parallel",)),
    )(*args)
    return zT, idx


def _make_table(p):
    args = [p['codebook'],
            p['dec0_w'], p['dec0_b'].reshape(1, 64),
            p['dec0_ln_s'].reshape(1, 64), p['dec0_ln_b'].reshape(1, 64),
            p['dec1_w'], p['dec1_b'].reshape(1, 128),
            p['dec1_ln_s'].reshape(1, 128), p['dec1_ln_b'].reshape(1, 128),
            p['dec2_w'], p['dec2_b'].reshape(1, N_OBS)]
    return pl.pallas_call(
        _table_body,
        in_specs=[_full(a.shape) for a in args],
        out_specs=_full((K, DEC_D)),
        out_shape=jax.ShapeDtypeStruct((K, DEC_D), jnp.float32),
    )(*args)


def _sc_gather(table, idx):
    info = plsc.get_sparse_core_info()
    nw = info.num_cores * info.num_subcores
    bw = B // nw
    mesh = plsc.VectorSubcoreMesh(core_axis_name="c", subcore_axis_name="s")

    @functools.partial(
        pl.kernel, mesh=mesh,
        out_type=jax.ShapeDtypeStruct((B, DEC_D), jnp.float32),
        scratch_types=[
            pltpu.VMEM((bw,), jnp.int32),
            pltpu.VMEM((bw, DEC_D), jnp.float32),
            pltpu.SemaphoreType.DMA,
        ],
    )
    def k(table_hbm, idx_hbm, out_hbm, idx_v, rows_v, sem):
        wid = lax.axis_index("s") * info.num_cores + lax.axis_index("c")
        base = wid * bw
        pltpu.sync_copy(idx_hbm.at[pl.ds(base, bw)], idx_v)
        pltpu.async_copy(table_hbm.at[idx_v], rows_v, sem).wait()
        pltpu.sync_copy(rows_v, out_hbm.at[pl.ds(base, bw)])

    return k(table, idx)


def kernel(input, params):
    xT = jnp.transpose(input, (1, 2, 0))
    zT, idx = _encode(xT, params)
    table = _make_table(params)
    g = _sc_gather(table, idx.reshape(B))
    quantized = g[:, :LAT]
    input_hat = g[:, LAT:LAT + N_OBS]
    z = zT.T
    return input_hat, quantized, z
